# baseline (speedup 1.0000x reference)
import functools

import jax
import jax.numpy as jnp
from jax import lax
from jax.experimental import pallas as pl
from jax.experimental.pallas import tpu as pltpu
from jax.experimental.pallas import tpu_sc as plsc

R = 64
N = 32768
L = 16
SEG = N // L
BITS = 8
BINS = 1 << BITS
NC = 2
NS = 16
NW = NC * NS
ROWS_PER_W = R // NW

def _fwd_mono_i(i):
    flip = lax.bitwise_or(lax.shift_right_arithmetic(i, 31),
                          jnp.int32(-2147483648))
    return lax.bitwise_xor(i, flip)


def _inv_mono(m_i32):
    flip = lax.bitwise_or(
        lax.bitwise_not(lax.shift_right_arithmetic(m_i32, 31)),
        jnp.int32(-2147483648))
    return lax.bitcast_convert_type(lax.bitwise_xor(m_i32, flip), jnp.float32)


K = 4
SUB = SEG // K
DU = 4


def _body(x_hbm, y_hbm, out_hbm, bufx, bufy, bufs, h0, h1, h2, h3, accv):
    hists = (h0, h1, h2, h3)
    wid = lax.axis_index("s") * NC + lax.axis_index("c")
    lane = lax.iota(jnp.int32, L)
    seg_base = lane * SEG
    ones_i = jnp.ones((L,), jnp.int32)
    zeros_i = jnp.zeros((L,), jnp.int32)
    mask = jnp.int32(BINS - 1)

    def digit(m, shift):
        return lax.bitwise_and(lax.shift_right_logical(m, shift), mask)

    def hist_index(m, shift):
        if shift >= 4:
            s = lax.shift_right_logical(m, shift - 4)
        else:
            s = lax.shift_left(m, 4 - shift)
        return lax.bitwise_or(
            lax.bitwise_and(s, jnp.int32((BINS - 1) << 4)), lane)

    def radix_pass(src, dst, shift, first):
        @plsc.parallel_loop(0, BINS, unroll=4)
        def z(i):
            sl = pl.ds(i * L, L)
            for hk in hists:
                hk[sl] = zeros_i

        @plsc.parallel_loop(0, SUB, unroll=2)
        def h(t):
            for k in range(K):
                sl = pl.ds((k * SUB + t) * L, L)
                m = lax.bitcast_convert_type(src[sl], jnp.int32)
                if first:
                    m = _fwd_mono_i(m)
                    src[sl] = lax.bitcast_convert_type(m, jnp.float32)
                plsc.addupdate_scatter(
                    hists[k], [hist_index(m, shift)], ones_i)

        def s(i, carry):
            sl = pl.ds(i * L, L)
            hv = [hk[sl] for hk in hists]
            tot = hv[0]
            for k in range(1, K):
                tot = tot + hv[k]
            off = jnp.cumsum(tot) - tot + carry
            for k in range(K):
                hists[k][sl] = off
                off = off + hv[k]
            return carry + jnp.sum(tot)
        lax.fori_loop(0, BINS, s, jnp.int32(0))

        @plsc.parallel_loop(0, SUB, unroll=2)
        def p(t):
            for k in range(K):
                sl = pl.ds((k * SUB + t) * L, L)
                v = src[sl]
                m = lax.bitcast_convert_type(v, jnp.int32)
                hidx = hist_index(m, shift)
                o = plsc.load_gather(hists[k], [hidx])
                plsc.addupdate_scatter(hists[k], [hidx], ones_i)
                paddr = lax.shift_left(
                    lax.bitwise_and(o, jnp.int32(SEG - 1)), 4) + \
                    lax.shift_right_logical(o, 11)
                plsc.store_scatter(dst, [paddr], v)

    def sort_inplace(buf):
        radix_pass(buf, bufs, 0, True)
        radix_pass(bufs, buf, 8, False)
        radix_pass(buf, bufs, 16, False)
        radix_pass(bufs, buf, 24, False)

    acc = jnp.zeros((L,), jnp.float32)
    for r in range(ROWS_PER_W):
        row = wid * ROWS_PER_W + r
        pltpu.sync_copy(x_hbm.at[row], bufx)
        pltpu.sync_copy(y_hbm.at[row], bufy)
        sort_inplace(bufx)
        sort_inplace(bufy)

        @plsc.parallel_loop(0, SEG // DU, unroll=2, carry=acc)
        def dacc(t, a):
            for u in range(DU):
                sl = pl.ds((t * DU + u) * L, L)
                fx = _inv_mono(lax.bitcast_convert_type(bufx[sl], jnp.int32))
                fy = _inv_mono(lax.bitcast_convert_type(bufy[sl], jnp.int32))
                diff = fx - fy
                a = a + diff * diff
            return a
        acc = dacc

    accv[...] = acc * jnp.float32(1.0 / N)
    pltpu.sync_copy(accv, out_hbm.at[wid])


@jax.jit
def _qag_partials(x, y):
    mesh = plsc.VectorSubcoreMesh(core_axis_name="c", subcore_axis_name="s")
    f = pl.kernel(
        _body,
        out_type=jax.ShapeDtypeStruct((NW, L), jnp.float32),
        mesh=mesh,
        compiler_params=pltpu.CompilerParams(
            needs_layout_passes=False,
            use_tc_tiling_on_sc=True,
        ),
        scratch_types=[
            pltpu.VMEM((N,), jnp.float32),
            pltpu.VMEM((N,), jnp.float32),
            pltpu.VMEM((N,), jnp.float32),
            pltpu.VMEM((BINS * L,), jnp.int32),
            pltpu.VMEM((BINS * L,), jnp.int32),
            pltpu.VMEM((BINS * L,), jnp.int32),
            pltpu.VMEM((BINS * L,), jnp.int32),
            pltpu.VMEM((L,), jnp.float32),
        ],
    )
    return f(x, y)


def kernel(x, y):
    return jnp.sum(_qag_partials(x, y))


# speedup vs baseline: 1.1716x; 1.1716x over previous
import functools

import jax
import jax.numpy as jnp
from jax import lax
from jax.experimental import pallas as pl
from jax.experimental.pallas import tpu as pltpu
from jax.experimental.pallas import tpu_sc as plsc

R = 64
N = 32768
L = 16
SEG = N // L
BITS = 8
BINS = 1 << BITS
NC = 2
NS = 16
NW = NC * NS
ROWS_PER_W = R // NW

def _fwd_mono_i(i):
    flip = lax.bitwise_or(lax.shift_right_arithmetic(i, 31),
                          jnp.int32(-2147483648))
    return lax.bitwise_xor(i, flip)


def _inv_mono(m_i32):
    flip = lax.bitwise_or(
        lax.bitwise_not(lax.shift_right_arithmetic(m_i32, 31)),
        jnp.int32(-2147483648))
    return lax.bitcast_convert_type(lax.bitwise_xor(m_i32, flip), jnp.float32)


K = 4
SUB = SEG // K
DU = 4


def _body(x_hbm, y_hbm, out_hbm, bufx, bufy, bufs, h0, h1, h2, h3, accv):
    hists = (h0, h1, h2, h3)
    wid = lax.axis_index("s") * NC + lax.axis_index("c")
    lane = lax.iota(jnp.int32, L)
    seg_base = lane * SEG
    ones_i = jnp.ones((L,), jnp.int32)
    zeros_i = jnp.zeros((L,), jnp.int32)
    mask = jnp.int32(BINS - 1)

    def digit(m, shift):
        return lax.bitwise_and(lax.shift_right_logical(m, shift), mask)

    def hist_index(m, shift):
        if shift >= 4:
            s = lax.shift_right_logical(m, shift - 4)
        else:
            s = lax.shift_left(m, 4 - shift)
        return lax.bitwise_or(
            lax.bitwise_and(s, jnp.int32((BINS - 1) << 4)), lane)

    def radix_pass(src, dst, shift, first):
        @plsc.parallel_loop(0, BINS, unroll=4)
        def z(i):
            sl = pl.ds(i * L, L)
            for hk in hists:
                hk[sl] = zeros_i

        @plsc.parallel_loop(0, SUB, unroll=2)
        def h(t):
            for k in range(K):
                sl = pl.ds((k * SUB + t) * L, L)
                m = lax.bitcast_convert_type(src[sl], jnp.int32)
                if first:
                    m = _fwd_mono_i(m)
                    src[sl] = lax.bitcast_convert_type(m, jnp.float32)
                plsc.addupdate_scatter(
                    hists[k], [hist_index(m, shift)], ones_i)

        @plsc.parallel_loop(0, BINS, unroll=2, carry=zeros_i)
        def s(i, vcarry):
            sl = pl.ds(i * L, L)
            hv = [hk[sl] for hk in hists]
            tot = hv[0]
            for k in range(1, K):
                tot = tot + hv[k]
            off = jnp.cumsum(tot) - tot + jnp.sum(vcarry)
            for k in range(K):
                hists[k][sl] = off
                off = off + hv[k]
            return vcarry + tot

        @plsc.parallel_loop(0, SUB, unroll=2)
        def p(t):
            for k in range(K):
                sl = pl.ds((k * SUB + t) * L, L)
                v = src[sl]
                m = lax.bitcast_convert_type(v, jnp.int32)
                hidx = hist_index(m, shift)
                o = plsc.load_gather(hists[k], [hidx])
                plsc.addupdate_scatter(hists[k], [hidx], ones_i)
                paddr = lax.shift_left(
                    lax.bitwise_and(o, jnp.int32(SEG - 1)), 4) + \
                    lax.shift_right_logical(o, 11)
                plsc.store_scatter(dst, [paddr], v)

    def sort_inplace(buf):
        radix_pass(buf, bufs, 0, True)
        radix_pass(bufs, buf, 8, False)
        radix_pass(buf, bufs, 16, False)
        radix_pass(bufs, buf, 24, False)

    acc = jnp.zeros((L,), jnp.float32)
    for r in range(ROWS_PER_W):
        row = wid * ROWS_PER_W + r
        pltpu.sync_copy(x_hbm.at[row], bufx)
        pltpu.sync_copy(y_hbm.at[row], bufy)
        sort_inplace(bufx)
        sort_inplace(bufy)

        @plsc.parallel_loop(0, SEG // DU, unroll=2, carry=acc)
        def dacc(t, a):
            for u in range(DU):
                sl = pl.ds((t * DU + u) * L, L)
                fx = _inv_mono(lax.bitcast_convert_type(bufx[sl], jnp.int32))
                fy = _inv_mono(lax.bitcast_convert_type(bufy[sl], jnp.int32))
                diff = fx - fy
                a = a + diff * diff
            return a
        acc = dacc

    accv[...] = acc * jnp.float32(1.0 / N)
    pltpu.sync_copy(accv, out_hbm.at[wid])


@jax.jit
def _qag_partials(x, y):
    mesh = plsc.VectorSubcoreMesh(core_axis_name="c", subcore_axis_name="s")
    f = pl.kernel(
        _body,
        out_type=jax.ShapeDtypeStruct((NW, L), jnp.float32),
        mesh=mesh,
        compiler_params=pltpu.CompilerParams(
            needs_layout_passes=False,
            use_tc_tiling_on_sc=True,
        ),
        scratch_types=[
            pltpu.VMEM((N,), jnp.float32),
            pltpu.VMEM((N,), jnp.float32),
            pltpu.VMEM((N,), jnp.float32),
            pltpu.VMEM((BINS * L,), jnp.int32),
            pltpu.VMEM((BINS * L,), jnp.int32),
            pltpu.VMEM((BINS * L,), jnp.int32),
            pltpu.VMEM((BINS * L,), jnp.int32),
            pltpu.VMEM((L,), jnp.float32),
        ],
    )
    return f(x, y)


def kernel(x, y):
    return jnp.sum(_qag_partials(x, y))


# speedup vs baseline: 1.2171x; 1.0388x over previous
import jax
import jax.numpy as jnp
from jax import lax
from jax.experimental import pallas as pl
from jax.experimental.pallas import tpu as pltpu
from jax.experimental.pallas import tpu_sc as plsc

R = 64
N = 32768
L = 16
SEG = N // L
BITS = 8
BINS = 1 << BITS
NC = 2
NS = 16
NW = NC * NS
ROWS_PER_W = R // NW
K = 2
SUB = SEG // K
CSET = K * BINS * L
DU = 4


def _fwd_mono_i(i):
    flip = lax.bitwise_or(lax.shift_right_arithmetic(i, 31),
                          jnp.int32(-2147483648))
    return lax.bitwise_xor(i, flip)


def _inv_mono(m_i32):
    flip = lax.bitwise_or(
        lax.bitwise_not(lax.shift_right_arithmetic(m_i32, 31)),
        jnp.int32(-2147483648))
    return lax.bitcast_convert_type(lax.bitwise_xor(m_i32, flip), jnp.float32)


def _body(x_hbm, y_hbm, out_hbm, bufx, bufy, bufs, pa, pb, accv):
    wid = lax.axis_index("s") * NC + lax.axis_index("c")
    lane = lax.iota(jnp.int32, L)
    ones_i = jnp.ones((L,), jnp.int32)
    zeros_i = jnp.zeros((L,), jnp.int32)

    def hist_index(m, shift):
        if shift >= 4:
            s = lax.shift_right_logical(m, shift - 4)
        else:
            s = lax.shift_left(m, 4 - shift)
        return lax.bitwise_or(
            lax.bitwise_and(s, jnp.int32((BINS - 1) << 4)), lane)

    def zero(cset):
        @plsc.parallel_loop(0, CSET // L, unroll=4)
        def z(i):
            cset[pl.ds(i * L, L)] = zeros_i

    def histogram_transform(src, cset):
        @plsc.parallel_loop(0, SUB, unroll=2)
        def h(t):
            for k in range(K):
                sl = pl.ds((k * SUB + t) * L, L)
                m = _fwd_mono_i(lax.bitcast_convert_type(src[sl], jnp.int32))
                src[sl] = lax.bitcast_convert_type(m, jnp.float32)
                idx = lax.bitwise_or(hist_index(m, 0), jnp.int32(k << 12))
                plsc.addupdate_scatter(cset, [idx], ones_i)

    def scan(cset):
        @plsc.parallel_loop(0, BINS, unroll=2, carry=zeros_i)
        def s(i, vcarry):
            sl = [pl.ds(k * BINS * L + i * L, L) for k in range(K)]
            hv = [cset[s_] for s_ in sl]
            tot = hv[0]
            for k in range(1, K):
                tot = tot + hv[k]
            off = jnp.cumsum(tot) - tot + jnp.sum(vcarry)
            for k in range(K):
                cset[sl[k]] = off
                off = off + hv[k]
            return vcarry + tot

    def permute(src, dst, ctr, shift, nxt, nshift):
        @plsc.parallel_loop(0, SUB, unroll=2)
        def p(t):
            for k in range(K):
                sl = pl.ds((k * SUB + t) * L, L)
                v = src[sl]
                m = lax.bitcast_convert_type(v, jnp.int32)
                hidx = lax.bitwise_or(hist_index(m, shift),
                                      jnp.int32(k << 12))
                o = plsc.load_gather(ctr, [hidx])
                plsc.addupdate_scatter(ctr, [hidx], ones_i)
                paddr = lax.shift_left(
                    lax.bitwise_and(o, jnp.int32(SEG - 1)), 4) + \
                    lax.shift_right_logical(o, 11)
                plsc.store_scatter(dst, [paddr], v)
                if nxt is not None:
                    d2 = lax.bitwise_and(
                        lax.shift_right_logical(m, nshift - 4),
                        jnp.int32((BINS - 1) << 4))
                    pos2 = lax.bitwise_or(
                        lax.bitwise_and(paddr, jnp.int32(L - 1)),
                        lax.shift_left(
                            lax.shift_right_logical(paddr, 14), 12))
                    plsc.addupdate_scatter(
                        nxt, [lax.bitwise_or(d2, pos2)], ones_i)

    def sort_inplace(buf):
        zero(pa)
        histogram_transform(buf, pa)
        scan(pa)
        zero(pb)
        permute(buf, bufs, pa, 0, pb, 8)
        scan(pb)
        zero(pa)
        permute(bufs, buf, pb, 8, pa, 16)
        scan(pa)
        zero(pb)
        permute(buf, bufs, pa, 16, pb, 24)
        scan(pb)
        permute(bufs, buf, pb, 24, None, 0)

    acc = jnp.zeros((L,), jnp.float32)
    for r in range(ROWS_PER_W):
        row = wid * ROWS_PER_W + r
        pltpu.sync_copy(x_hbm.at[row], bufx)
        pltpu.sync_copy(y_hbm.at[row], bufy)
        sort_inplace(bufx)
        sort_inplace(bufy)

        @plsc.parallel_loop(0, SEG // DU, unroll=2, carry=acc)
        def dacc(t, a):
            for u in range(DU):
                sl = pl.ds((t * DU + u) * L, L)
                fx = _inv_mono(lax.bitcast_convert_type(bufx[sl], jnp.int32))
                fy = _inv_mono(lax.bitcast_convert_type(bufy[sl], jnp.int32))
                diff = fx - fy
                a = a + diff * diff
            return a
        acc = dacc

    accv[...] = acc * jnp.float32(1.0 / N)
    pltpu.sync_copy(accv, out_hbm.at[wid])


@jax.jit
def _qag_partials(x, y):
    mesh = plsc.VectorSubcoreMesh(core_axis_name="c", subcore_axis_name="s")
    f = pl.kernel(
        _body,
        out_type=jax.ShapeDtypeStruct((NW, L), jnp.float32),
        mesh=mesh,
        compiler_params=pltpu.CompilerParams(
            needs_layout_passes=False,
            use_tc_tiling_on_sc=True,
        ),
        scratch_types=[
            pltpu.VMEM((N,), jnp.float32),
            pltpu.VMEM((N,), jnp.float32),
            pltpu.VMEM((N,), jnp.float32),
            pltpu.VMEM((CSET,), jnp.int32),
            pltpu.VMEM((CSET,), jnp.int32),
            pltpu.VMEM((L,), jnp.float32),
        ],
    )
    return f(x, y)


def kernel(x, y):
    return jnp.sum(_qag_partials(x, y))
